# baseline (speedup 1.0000x reference)
import jax
import jax.numpy as jnp
from jax.experimental import pallas as pl
from jax.experimental.pallas import tpu as pltpu

IN_F = 10
OUT_F = 5
SPLIT = 4
TB = 4096


def _linear_kernel(x0, x1, x2, x3, w_ref, b_ref, o_ref):
    w = w_ref[...]
    b = b_ref[...]
    for k, x_ref in enumerate((x0, x1, x2, x3)):
        acc = jnp.dot(x_ref[...], w, preferred_element_type=jnp.float32)
        o_ref[k * TB:(k + 1) * TB, :] = (acc + b).astype(o_ref.dtype)


@jax.jit
def _forward(x, w_packed, b_packed):
    B, in_f = x.shape
    assert in_f == IN_F and B % (SPLIT * TB) == 0

    w = w_packed[:, :OUT_F]
    b = b_packed[:, :OUT_F]

    def mk(k):
        return pl.BlockSpec((TB, IN_F), lambda i, k=k: (SPLIT * i + k, 0))

    out = pl.pallas_call(
        _linear_kernel,
        out_shape=jax.ShapeDtypeStruct((B, OUT_F), x.dtype),
        grid=(B // (SPLIT * TB),),
        in_specs=[mk(0), mk(1), mk(2), mk(3),
                  pl.BlockSpec((IN_F, OUT_F), lambda i: (0, 0)),
                  pl.BlockSpec((1, OUT_F), lambda i: (0, 0))],
        out_specs=pl.BlockSpec((SPLIT * TB, OUT_F), lambda i: (i, 0)),
        compiler_params=pltpu.CompilerParams(
            dimension_semantics=("parallel",),
        ),
    )(x, x, x, x, w, b)
    return out


def kernel(x, w_packed, b_packed):
    return _forward(x, w_packed, b_packed)
